# Initial kernel scaffold; baseline (speedup 1.0000x reference)
import jax
import jax.numpy as jnp
from jax.experimental import pallas as pl


def kernel(x, A, weight, bias, mask):
    raise NotImplementedError("write your pallas kernel here")


# speedup vs baseline: 1.0036x; 1.0036x over previous
import jax
import jax.numpy as jnp
from jax.experimental import pallas as pl
from jax.experimental.pallas import tpu as pltpu


def _ctg_body(x_ref, w_ref, b_ref, m_ref, o_ref):
    xb = x_ref[0].astype(jnp.bfloat16)
    acc = jax.lax.dot_general(
        w_ref[...], xb,
        dimension_numbers=(((1,), (0,)), ((), ())),
        preferred_element_type=jnp.float32)
    o_ref[0] = (acc + b_ref[...]) * m_ref[0]


def kernel(x, A, weight, bias, mask, *, tv_tile=2048):
    N, C_in, T, V = x.shape
    C_out = weight.shape[0]
    TV = T * V
    if TV % tv_tile != 0:
        tv_tile = TV
    grid = (N, TV // tv_tile)

    x3 = x.reshape(N, C_in, TV)
    w2 = weight.reshape(C_out, C_in).astype(jnp.bfloat16)
    b2 = bias.reshape(C_out, 1).astype(jnp.float32)
    m3 = mask.reshape(N, 1, TV).astype(x.dtype)

    out3 = pl.pallas_call(
        _ctg_body,
        out_shape=jax.ShapeDtypeStruct((N, C_out, TV), x.dtype),
        grid=grid,
        in_specs=[
            pl.BlockSpec((1, C_in, tv_tile), lambda n, j: (n, 0, j)),
            pl.BlockSpec((C_out, C_in), lambda n, j: (0, 0)),
            pl.BlockSpec((C_out, 1), lambda n, j: (0, 0)),
            pl.BlockSpec((1, 1, tv_tile), lambda n, j: (n, 0, j)),
        ],
        out_specs=pl.BlockSpec((1, C_out, tv_tile), lambda n, j: (n, 0, j)),
        compiler_params=pltpu.CompilerParams(
            dimension_semantics=("parallel", "parallel")),
        cost_estimate=pl.CostEstimate(
            flops=2 * N * C_out * C_in * TV,
            transcendentals=0,
            bytes_accessed=4 * (N * C_in * TV + N * C_out * TV + N * TV)),
    )(x3, w2, b2, m3)

    return out3.reshape(N, C_out, T, V), A
